# baseline (speedup 1.0000x reference)
import functools
import jax
import jax.numpy as jnp
from jax import lax
from jax.experimental import pallas as pl
from jax.experimental.pallas import tpu as pltpu
from jax.experimental.pallas import tpu_sc as plsc

_S, _B, _D = 8192, 4, 1024
_BD = _B * _D
_NC, _NS = 2, 16
_NW = _NC * _NS
_ROWS = _S // _NW
_R = 4
_NCHUNK = _ROWS // _R
_SLOTS = 4
_LANES = 16


def _build(interpret=False):
    mesh = plsc.VectorSubcoreMesh(
        core_axis_name="c", subcore_axis_name="s",
        num_cores=_NC, num_subcores=_NS)

    @functools.partial(
        pl.kernel,
        out_type=jax.ShapeDtypeStruct((_S, _BD), jnp.float32),
        mesh=mesh,
        scratch_types=[
            pltpu.VMEM((_SLOTS, _R, _BD), jnp.float32),
            pltpu.VMEM((_SLOTS, _R, _D), jnp.float32),
            pltpu.SemaphoreType.DMA((_SLOTS,)),
            pltpu.SemaphoreType.DMA((_SLOTS,)),
        ],
        interpret=interpret,
    )
    def sc_add(x_hbm, pe_hbm, out_hbm, xv, pev, insem, outsem):
        wid = lax.axis_index("s") * _NC + lax.axis_index("c")
        base = wid * _ROWS

        def in_copies(i, slot):
            row = base + i * _R
            return (
                pltpu.make_async_copy(
                    x_hbm.at[pl.ds(row, _R)], xv.at[slot], insem.at[slot]),
                pltpu.make_async_copy(
                    pe_hbm.at[pl.ds(row, _R)], pev.at[slot], insem.at[slot]),
            )

        def out_copy(i, slot):
            row = base + i * _R
            return pltpu.make_async_copy(
                xv.at[slot], out_hbm.at[pl.ds(row, _R)], outsem.at[slot])

        def start_in(i, slot):
            a, b = in_copies(i, slot)
            a.start()
            b.start()

        def wait_in(i, slot):
            a, b = in_copies(i, slot)
            a.wait()
            b.wait()

        for s in range(_SLOTS - 1):
            start_in(s, s)

        def compute(slot):
            @pl.loop(0, _D // _LANES)
            def _(c):
                off = c * _LANES
                for r in range(_R):
                    p = pev[slot, r, pl.ds(off, _LANES)]
                    for q in range(_B):
                        col = q * _D
                        xv[slot, r, pl.ds(col + off, _LANES)] = (
                            xv[slot, r, pl.ds(col + off, _LANES)] + p)

        @pl.loop(0, _NCHUNK, step=_SLOTS)
        def _(g):
            for b in range(_SLOTS):
                i = g + b
                wait_in(i, b)
                compute(b)
                out_copy(i, b).start()
                nslot = (b + _SLOTS - 1) % _SLOTS
                nxt = i + _SLOTS - 1

                @pl.when(nxt < _NCHUNK)
                def _():
                    @pl.when(i >= 1)
                    def _():
                        out_copy(i - 1, nslot).wait()

                    start_in(nxt, nslot)

        for s in range(_SLOTS):
            out_copy(_NCHUNK - _SLOTS + s, s).wait()

    return sc_add


_sc_impl = _build()


def kernel(x, position_embeddings):
    S, B, D = x.shape
    x2 = x.reshape(S, B * D)
    out = _sc_impl(x2, position_embeddings[:S])
    return out.reshape(S, B, D)


# speedup vs baseline: 1.0329x; 1.0329x over previous
import jax
import jax.numpy as jnp
from jax.experimental import pallas as pl

_BS = 256


def _body(x_ref, pe_ref, o_ref):
    pe = pe_ref[...]
    o_ref[...] = x_ref[...] + jnp.concatenate([pe, pe, pe, pe], axis=1)


def kernel(x, position_embeddings):
    S, B, D = x.shape
    x2 = x.reshape(S, B * D)
    out = pl.pallas_call(
        _body,
        grid=(S // _BS,),
        in_specs=[
            pl.BlockSpec((_BS, B * D), lambda i: (i, 0)),
            pl.BlockSpec((_BS, D), lambda i: (i, 0)),
        ],
        out_specs=pl.BlockSpec((_BS, B * D), lambda i: (i, 0)),
        out_shape=jax.ShapeDtypeStruct((S, B * D), x.dtype),
    )(x2, position_embeddings[:S])
    return out.reshape(S, B, D)


# speedup vs baseline: 3.9061x; 3.7818x over previous
import jax
import jax.numpy as jnp
from jax.experimental import pallas as pl

_BS = 256


def _body(x_ref, pe_ref, o_ref):
    o_ref[...] = x_ref[...] + pe_ref[...][:, None, :]


def kernel(x, position_embeddings):
    S, B, D = x.shape
    return pl.pallas_call(
        _body,
        grid=(S // _BS,),
        in_specs=[
            pl.BlockSpec((_BS, B, D), lambda i: (i, 0, 0)),
            pl.BlockSpec((_BS, D), lambda i: (i, 0)),
        ],
        out_specs=pl.BlockSpec((_BS, B, D), lambda i: (i, 0, 0)),
        out_shape=jax.ShapeDtypeStruct((S, B, D), x.dtype),
    )(x, position_embeddings[:S])


# speedup vs baseline: 3.9825x; 1.0196x over previous
import jax
import jax.numpy as jnp
from jax.experimental import pallas as pl

_BS = 512


def _body(x_ref, pe_ref, o_ref):
    o_ref[...] = x_ref[...] + pe_ref[...][:, None, :]


def kernel(x, position_embeddings):
    S, B, D = x.shape
    return pl.pallas_call(
        _body,
        grid=(S // _BS,),
        in_specs=[
            pl.BlockSpec((_BS, B, D), lambda i: (i, 0, 0)),
            pl.BlockSpec((_BS, D), lambda i: (i, 0)),
        ],
        out_specs=pl.BlockSpec((_BS, B, D), lambda i: (i, 0, 0)),
        out_shape=jax.ShapeDtypeStruct((S, B, D), x.dtype),
    )(x, position_embeddings[:S])
